# baseline (speedup 1.0000x reference)
import functools

import jax
import jax.numpy as jnp
from jax import lax
from jax.experimental import pallas as pl
from jax.experimental.pallas import tpu as pltpu
from jax.experimental.pallas import tpu_sc as plsc

_B, _Q, _D = 16, 8000, 256
_NCLS, _QPC = 200, 40
_OW = _NCLS + 1
_BQ = _B * _Q
_NC, _NS = 2, 16
_NW = _NC * _NS
_RPT = _BQ // _NW
_CPT = _RPT // _QPC
_CH = 2 * _QPC
_NCHUNK = _RPT // _CH
_NT = _D // 16
_OBW = _CH * _OW


def _rsqrt16(x):
    i = plsc.bitcast(x, jnp.int32)
    y = plsc.bitcast(jnp.int32(0x5F3759DF) - (i >> 1), jnp.float32)
    for _ in range(3):
        y = y * (1.5 - 0.5 * x * y * y)
    return y


_mesh = plsc.VectorSubcoreMesh(core_axis_name="c", subcore_axis_name="s")


@functools.partial(
    pl.kernel,
    out_type=jax.ShapeDtypeStruct((_B, _Q, _OW), jnp.float32),
    mesh=_mesh,
    compiler_params=pltpu.CompilerParams(needs_layout_passes=False),
    scratch_types=[
        pltpu.VMEM(((_CPT + 1) * _D,), jnp.float32),
        pltpu.VMEM((_CH, _D), jnp.float32),
        pltpu.VMEM((_CH, _D), jnp.float32),
        pltpu.VMEM((_CH * 16,), jnp.float32),
        pltpu.VMEM((_CH * 16,), jnp.float32),
        pltpu.VMEM((_CH * 16,), jnp.float32),
        pltpu.VMEM((_CH, _OW), jnp.float32),
        pltpu.VMEM((_CH, _OW), jnp.float32),
        pltpu.SemaphoreType.DMA,
        pltpu.SemaphoreType.DMA,
        pltpu.SemaphoreType.DMA,
        pltpu.SemaphoreType.DMA,
    ],
)
def _head_kernel(qf_hbm, ce_hbm, out_hbm, cev, qb0, qb1, aqq, adc, adb,
                 ob0, ob1, qsem0, qsem1, osem0, osem1):
    wid = lax.axis_index("s") * _NC + lax.axis_index("c")
    b = wid // 2
    half = wid % 2
    q0 = half * _RPT
    cls_base = half * _CPT
    iota = lax.iota(jnp.int32, 16)
    ninf = jnp.full((16,), -jnp.inf, jnp.float32)

    pltpu.sync_copy(ce_hbm.at[pl.ds(cls_base * _D, _CPT * _D)],
                    cev.at[pl.ds(0, _CPT * _D)])
    pltpu.sync_copy(ce_hbm.at[pl.ds(_NCLS * _D, _D)],
                    cev.at[pl.ds(_CPT * _D, _D)])

    bg = [cev[pl.ds(_CPT * _D + t * 16, 16)] for t in range(_NT)]
    bacc = bg[0] * bg[0]
    for t in range(1, _NT):
        bacc = bacc + bg[t] * bg[t]
    rbg = _rsqrt16(jnp.maximum(jnp.full((16,), jnp.sum(bacc)), 1e-24))

    for ob in (ob0, ob1):
        for r in range(_CH):
            for t in range(12):
                ob[r, pl.ds(t * 16, 16)] = ninf
            plsc.store_scatter(ob, [jnp.full((16,), r, jnp.int32), 192 + iota],
                               ninf, mask=iota < 9)

    def q_copy(k, qb, qsem):
        off = pl.multiple_of(q0 + k * _CH, 16)
        return pltpu.async_copy(qf_hbm.at[b, pl.ds(off, _CH)], qb, qsem)

    q_copy(0, qb0, qsem0)
    q_copy(1, qb1, qsem1)

    def class_prep(cls_local):
        koff = pl.multiple_of(cls_local * _D, 256)
        ce_t = [cev[pl.ds(koff + t * 16, 16)] for t in range(_NT)]
        eacc = ce_t[0] * ce_t[0]
        for t in range(1, _NT):
            eacc = eacc + ce_t[t] * ce_t[t]
        rce = _rsqrt16(jnp.maximum(jnp.full((16,), jnp.sum(eacc)), 1e-24))
        return ce_t, rce

    def half_rows(qb, ce_t, row0):
        def row_body(r, carry):
            qv = [qb[r, pl.ds(t * 16, 16)] for t in range(4)]
            aq = [qv[i] * qv[i] for i in range(4)]
            ac = [qv[i] * ce_t[i] for i in range(4)]
            ab = [qv[i] * bg[i] for i in range(4)]
            for t in range(4, _NT, 4):
                qv = [qb[r, pl.ds((t + i) * 16, 16)] for i in range(4)]
                for i in range(4):
                    aq[i] = aq[i] + qv[i] * qv[i]
                    ac[i] = ac[i] + qv[i] * ce_t[t + i]
                    ab[i] = ab[i] + qv[i] * bg[t + i]
            soff = pl.multiple_of(r * 16, 16)
            aqq[pl.ds(soff, 16)] = plsc.cumsum((aq[0] + aq[1]) + (aq[2] + aq[3]))
            adc[pl.ds(soff, 16)] = plsc.cumsum((ac[0] + ac[1]) + (ac[2] + ac[3]))
            adb[pl.ds(soff, 16)] = plsc.cumsum((ab[0] + ab[1]) + (ab[2] + ab[3]))
            return carry
        lax.fori_loop(row0, row0 + _QPC, row_body, 0, unroll=2)

    def chunk_body(k, qb, ob, qsem, osem):
        c0 = cls_base + 2 * k
        pltpu.make_async_copy(qf_hbm.at[b, pl.ds(q0, _CH)], qb, qsem).wait()
        @pl.when(k >= 2)
        def _():
            pltpu.make_async_copy(
                ob, out_hbm.at[b, pl.ds(q0, _CH)], osem).wait()
            for g in range(5):
                rvec = g * 16 + iota
                cold = jnp.where(rvec < _QPC, c0 - 4, c0 - 3)
                plsc.store_scatter(ob, [rvec, cold], ninf)

        ce_a, rce_a = class_prep(2 * k)
        ce_b, rce_b = class_prep(2 * k + 1)
        half_rows(qb, ce_a, 0)
        half_rows(qb, ce_b, _QPC)

        for g in range(5):
            rvec = g * 16 + iota
            fb = rvec * 16 + 15
            qq = plsc.load_gather(aqq, [fb])
            dc = plsc.load_gather(adc, [fb])
            db = plsc.load_gather(adb, [fb])
            rq = _rsqrt16(jnp.maximum(qq, 1e-24))
            in_a = rvec < _QPC
            rce = jnp.where(in_a, rce_a, rce_b)
            simc = dc * rq * rce
            simb = db * rq * rbg
            cvec = jnp.where(in_a, c0, c0 + 1)
            plsc.store_scatter(ob, [rvec, cvec], simc)
            plsc.store_scatter(ob, [rvec, jnp.full((16,), _NCLS, jnp.int32)],
                               simb)

        pltpu.async_copy(ob, out_hbm.at[b, pl.ds(q0 + k * _CH, _CH)], osem)

        @pl.when(k + 2 < _NCHUNK)
        def _():
            q_copy(k + 2, qb, qsem)

    def outer(kk, carry):
        chunk_body(kk * 2, qb0, ob0, qsem0, osem0)
        chunk_body(kk * 2 + 1, qb1, ob1, qsem1, osem1)
        return carry

    lax.fori_loop(0, _NCHUNK // 2, outer, 0)

    pltpu.make_async_copy(ob0, out_hbm.at[b, pl.ds(q0, _CH)], osem0).wait()
    pltpu.make_async_copy(ob1, out_hbm.at[b, pl.ds(q0, _CH)], osem1).wait()


@jax.jit
def kernel(query_features, class_embeddings):
    return _head_kernel(query_features, class_embeddings.reshape(-1))


# speedup vs baseline: 1.0458x; 1.0458x over previous
import functools

import jax
import jax.numpy as jnp
from jax import lax
from jax.experimental import pallas as pl
from jax.experimental.pallas import tpu as pltpu
from jax.experimental.pallas import tpu_sc as plsc

_B, _Q, _D = 16, 8000, 256
_NCLS, _QPC = 200, 40
_OW = _NCLS + 1
_BQ = _B * _Q
_NC, _NS = 2, 16
_NW = _NC * _NS
_RPT = _BQ // _NW
_CPT = _RPT // _QPC
_CH = 2 * _QPC
_NCHUNK = _RPT // _CH
_NT = _D // 16
_OBW = _CH * _OW


def _rsqrt16(x):
    i = plsc.bitcast(x, jnp.int32)
    y = plsc.bitcast(jnp.int32(0x5F3759DF) - (i >> 1), jnp.float32)
    for _ in range(3):
        y = y * (1.5 - 0.5 * x * y * y)
    return y


_mesh = plsc.VectorSubcoreMesh(core_axis_name="c", subcore_axis_name="s")


@functools.partial(
    pl.kernel,
    out_type=jax.ShapeDtypeStruct((_B, _Q, _OW), jnp.float32),
    mesh=_mesh,
    compiler_params=pltpu.CompilerParams(needs_layout_passes=False),
    scratch_types=[
        pltpu.VMEM(((_CPT + 1) * _D,), jnp.float32),
        pltpu.VMEM((_CH, _D), jnp.float32),
        pltpu.VMEM((_CH, _D), jnp.float32),
        pltpu.VMEM((_CH * 16,), jnp.float32),
        pltpu.VMEM((_CH * 16,), jnp.float32),
        pltpu.VMEM((_CH * 16,), jnp.float32),
        pltpu.VMEM((_CH, _OW), jnp.float32),
        pltpu.VMEM((_CH, _OW), jnp.float32),
        pltpu.SemaphoreType.DMA,
        pltpu.SemaphoreType.DMA,
        pltpu.SemaphoreType.DMA,
        pltpu.SemaphoreType.DMA,
    ],
)
def _head_kernel(qf_hbm, ce_hbm, out_hbm, cev, qb0, qb1, aqq, adc, adb,
                 ob0, ob1, qsem0, qsem1, osem0, osem1):
    wid = lax.axis_index("s") * _NC + lax.axis_index("c")
    b = wid // 2
    half = wid % 2
    q0 = half * _RPT
    cls_base = half * _CPT
    iota = lax.iota(jnp.int32, 16)
    ninf = jnp.full((16,), -jnp.inf, jnp.float32)

    pltpu.sync_copy(ce_hbm.at[pl.ds(cls_base * _D, _CPT * _D)],
                    cev.at[pl.ds(0, _CPT * _D)])
    pltpu.sync_copy(ce_hbm.at[pl.ds(_NCLS * _D, _D)],
                    cev.at[pl.ds(_CPT * _D, _D)])

    bg = [cev[pl.ds(_CPT * _D + t * 16, 16)] for t in range(_NT)]
    bacc = bg[0] * bg[0]
    for t in range(1, _NT):
        bacc = bacc + bg[t] * bg[t]
    rbg = _rsqrt16(jnp.maximum(jnp.full((16,), jnp.sum(bacc)), 1e-24))

    for ob in (ob0, ob1):
        for r in range(_CH):
            for t in range(12):
                ob[r, pl.ds(t * 16, 16)] = ninf
            plsc.store_scatter(ob, [jnp.full((16,), r, jnp.int32), 192 + iota],
                               ninf, mask=iota < 9)

    def q_copy(k, qb, qsem):
        off = pl.multiple_of(q0 + k * _CH, 16)
        return pltpu.async_copy(qf_hbm.at[b, pl.ds(off, _CH)], qb, qsem)

    q_copy(0, qb0, qsem0)
    q_copy(1, qb1, qsem1)

    def class_prep(cls_local):
        koff = pl.multiple_of(cls_local * _D, 256)
        ce_t = [cev[pl.ds(koff + t * 16, 16)] for t in range(_NT)]
        eacc = ce_t[0] * ce_t[0]
        for t in range(1, _NT):
            eacc = eacc + ce_t[t] * ce_t[t]
        rce = _rsqrt16(jnp.maximum(jnp.full((16,), jnp.sum(eacc)), 1e-24))
        return ce_t, rce

    def half_rows(qb, ce_t, row0):
        def row_body(r, carry):
            qa = qb[r, pl.ds(0, 16)]
            qc = qb[r, pl.ds(16, 16)]
            aq0, aq1 = qa * qa, qc * qc
            ac0, ac1 = qa * ce_t[0], qc * ce_t[1]
            ab0, ab1 = qa * bg[0], qc * bg[1]
            for t in range(2, _NT, 2):
                qa = qb[r, pl.ds(t * 16, 16)]
                qc = qb[r, pl.ds((t + 1) * 16, 16)]
                aq0, aq1 = aq0 + qa * qa, aq1 + qc * qc
                ac0, ac1 = ac0 + qa * ce_t[t], ac1 + qc * ce_t[t + 1]
                ab0, ab1 = ab0 + qa * bg[t], ab1 + qc * bg[t + 1]
            soff = pl.multiple_of(r * 16, 16)
            aqq[pl.ds(soff, 16)] = plsc.cumsum(aq0 + aq1)
            adc[pl.ds(soff, 16)] = plsc.cumsum(ac0 + ac1)
            adb[pl.ds(soff, 16)] = plsc.cumsum(ab0 + ab1)
            return carry
        lax.fori_loop(row0, row0 + _QPC, row_body, 0, unroll=4)

    def chunk_body(k, qb, ob, qsem, osem):
        c0 = cls_base + 2 * k
        ce_a, rce_a = class_prep(2 * k)
        ce_b, rce_b = class_prep(2 * k + 1)
        pltpu.make_async_copy(qf_hbm.at[b, pl.ds(q0, _CH)], qb, qsem).wait()
        @pl.when(k >= 2)
        def _():
            pltpu.make_async_copy(
                ob, out_hbm.at[b, pl.ds(q0, _CH)], osem).wait()
            for g in range(5):
                rvec = g * 16 + iota
                cold = jnp.where(rvec < _QPC, c0 - 4, c0 - 3)
                plsc.store_scatter(ob, [rvec, cold], ninf)

        half_rows(qb, ce_a, 0)
        half_rows(qb, ce_b, _QPC)

        @pl.when(k + 2 < _NCHUNK)
        def _():
            q_copy(k + 2, qb, qsem)

        for g in range(5):
            rvec = g * 16 + iota
            fb = rvec * 16 + 15
            qq = plsc.load_gather(aqq, [fb])
            dc = plsc.load_gather(adc, [fb])
            db = plsc.load_gather(adb, [fb])
            rq = _rsqrt16(jnp.maximum(qq, 1e-24))
            in_a = rvec < _QPC
            rce = jnp.where(in_a, rce_a, rce_b)
            simc = dc * rq * rce
            simb = db * rq * rbg
            cvec = jnp.where(in_a, c0, c0 + 1)
            plsc.store_scatter(ob, [rvec, cvec], simc)
            plsc.store_scatter(ob, [rvec, jnp.full((16,), _NCLS, jnp.int32)],
                               simb)

        pltpu.async_copy(ob, out_hbm.at[b, pl.ds(q0 + k * _CH, _CH)], osem)

    def outer(kk, carry):
        chunk_body(kk * 2, qb0, ob0, qsem0, osem0)
        chunk_body(kk * 2 + 1, qb1, ob1, qsem1, osem1)
        return carry

    lax.fori_loop(0, _NCHUNK // 2, outer, 0)

    pltpu.make_async_copy(ob0, out_hbm.at[b, pl.ds(q0, _CH)], osem0).wait()
    pltpu.make_async_copy(ob1, out_hbm.at[b, pl.ds(q0, _CH)], osem1).wait()


@jax.jit
def kernel(query_features, class_embeddings):
    return _head_kernel(query_features, class_embeddings.reshape(-1))
